# baseline (speedup 1.0000x reference)
import functools

import jax
import jax.numpy as jnp
from jax import lax
from jax.experimental import pallas as pl
from jax.experimental.pallas import tpu as pltpu
from jax.experimental.pallas import tpu_sc as plsc

_N = 10000
_E = 320000
_D = 128
_DE = 16

_NC = 2
_NS = 16
_NW = _NC * _NS
_NPAD = 10240
_RPT = _NPAD // _NS

_E1 = 192000
_E2 = _E - _E1

_mesh = plsc.VectorSubcoreMesh(core_axis_name="c", subcore_axis_name="s")


def _vadd_chunk(k, bufa, bufb, bufg):
    def row(r, carry):
        for g in range(_D // 16):
            sl = (r, pl.ds(g * 16, 16))
            bufg[sl] = bufa[sl] + bufb[sl]
        return carry

    lax.fori_loop(0, k, row, 0)


def _make_gather(e_, k_, eoff_):
    epw = e_ // _NW
    nch = epw // k_
    assert epw % k_ == 0 and nch % 2 == 1 and k_ % 8 == 0 and k_ <= 128
    last = nch - 1

    def body(a_hbm, b_hbm, src_hbm, dst_hbm, g_hbm,
             ia0, ib0, ia1, ib1, a0, b0, g0, a1, b1, g1,
             sia0, sib0, sia1, sib1, sga0, sgb0, sga1, sgb1, sw0, sw1):
        wid = lax.axis_index("s") * _NC + lax.axis_index("c")
        base = wid * epw
        ibase = eoff_ + base

        sets = ((ia0, ib0, a0, b0, g0, sia0, sib0, sga0, sgb0, sw0),
                (ia1, ib1, a1, b1, g1, sia1, sib1, sga1, sgb1, sw1))

        for s in (0, 1):
            ia, ib, ba, bb, bg, sia, sib, sga, sgb, sw = sets[s]
            off = ibase + s * k_
            pltpu.sync_copy(src_hbm.at[pl.ds(off, k_)], ia)
            pltpu.sync_copy(dst_hbm.at[pl.ds(off, k_)], ib)
            pltpu.async_copy(a_hbm.at[ia], ba, sga)
            pltpu.async_copy(b_hbm.at[ib], bb, sgb)

        def step(j, carry):
            for s in (0, 1):
                ia, ib, ba, bb, bg, sia, sib, sga, sgb, sw = sets[s]
                c = 2 * j + s
                off = base + c * k_
                cn = jnp.minimum(c + 2, last)
                offn = ibase + cn * k_
                pltpu.make_async_copy(a_hbm.at[ia], ba, sga).wait()
                pltpu.make_async_copy(b_hbm.at[ib], bb, sgb).wait()
                pltpu.async_copy(src_hbm.at[pl.ds(offn, k_)], ia, sia)
                pltpu.async_copy(dst_hbm.at[pl.ds(offn, k_)], ib, sib)

                @pl.when(c >= 2)
                def _():
                    offp = base + (c - 2) * k_
                    pltpu.make_async_copy(
                        bg, g_hbm.at[pl.ds(offp, k_)], sw).wait()

                _vadd_chunk(k_, ba, bb, bg)
                pltpu.async_copy(bg, g_hbm.at[pl.ds(off, k_)], sw)
                pltpu.make_async_copy(src_hbm.at[pl.ds(offn, k_)], ia, sia).wait()
                pltpu.make_async_copy(dst_hbm.at[pl.ds(offn, k_)], ib, sib).wait()
                pltpu.async_copy(a_hbm.at[ia], ba, sga)
                pltpu.async_copy(b_hbm.at[ib], bb, sgb)
            return carry

        lax.fori_loop(0, (nch - 1) // 2, step, 0)

        ia, ib, ba, bb, bg, sia, sib, sga, sgb, sw = sets[0]
        pltpu.make_async_copy(a_hbm.at[ia], ba, sga).wait()
        pltpu.make_async_copy(b_hbm.at[ib], bb, sgb).wait()
        pltpu.make_async_copy(
            bg, g_hbm.at[pl.ds(base + (last - 2) * k_, k_)], sw).wait()
        _vadd_chunk(k_, ba, bb, bg)
        pltpu.sync_copy(bg, g_hbm.at[pl.ds(base + last * k_, k_)])

        ia, ib, ba, bb, bg, sia, sib, sga, sgb, sw = sets[1]
        pltpu.make_async_copy(a_hbm.at[ia], ba, sga).wait()
        pltpu.make_async_copy(b_hbm.at[ib], bb, sgb).wait()
        pltpu.make_async_copy(
            bg, g_hbm.at[pl.ds(base + (last - 1) * k_, k_)], sw).wait()

    return pl.kernel(
        body,
        mesh=_mesh,
        out_type=[jax.ShapeDtypeStruct((e_, _D), jnp.float32)],
        scratch_types=(
            [pltpu.VMEM((k_,), jnp.int32)] * 4
            + [pltpu.VMEM((k_, _D), jnp.float32)] * 6
            + [pltpu.SemaphoreType.DMA] * 10
        ),
    )


_gather1 = _make_gather(_E1, 80, 0)
_gather2 = _make_gather(_E2, 32, _E1)


def _make_scatter(e_, k_, eoff_):
    epw = e_ // _NW
    epc = e_ // _NC
    nch = epw // k_
    assert epw % k_ == 0 and nch % 2 == 1 and k_ % 8 == 0 and k_ <= 128
    last = nch - 1

    def body(h_hbm, dst_hbm, zeros_hbm, out_hbm,
             idx0, idx1, hb0, hb1, acc, si0, si1, sh0, sh1):
        cid = lax.axis_index("c")
        sid = lax.axis_index("s")
        r0 = sid * _RPT
        pltpu.sync_copy(zeros_hbm.at[pl.ds(r0, _RPT)], acc.at[pl.ds(r0, _RPT)])

        base = cid * epc + sid * epw
        ibase = eoff_ + base

        def start_loads(c, idxb, hbufb, semi, semh):
            pltpu.async_copy(
                dst_hbm.at[pl.ds(ibase + c * k_, k_)], idxb, semi)
            pltpu.async_copy(h_hbm.at[pl.ds(base + c * k_, k_)], hbufb, semh)

        def wait_loads(c, idxb, hbufb, semi, semh):
            pltpu.make_async_copy(
                dst_hbm.at[pl.ds(ibase + c * k_, k_)], idxb, semi).wait()
            pltpu.make_async_copy(
                h_hbm.at[pl.ds(base + c * k_, k_)], hbufb, semh).wait()

        plsc.subcore_barrier()
        start_loads(0, idx0, hb0, si0, sh0)
        start_loads(1, idx1, hb1, si1, sh1)

        def step(j, carry):
            c0 = 2 * j
            wait_loads(c0, idx0, hb0, si0, sh0)
            pltpu.sync_copy(hb0, acc.at[idx0], add=True)
            start_loads(jnp.minimum(c0 + 2, last), idx0, hb0, si0, sh0)
            c1 = c0 + 1
            wait_loads(c1, idx1, hb1, si1, sh1)
            pltpu.sync_copy(hb1, acc.at[idx1], add=True)
            start_loads(jnp.minimum(c1 + 2, last), idx1, hb1, si1, sh1)
            return carry

        lax.fori_loop(0, (nch - 1) // 2, step, 0)
        wait_loads(last, idx0, hb0, si0, sh0)
        wait_loads(last, idx1, hb1, si1, sh1)
        pltpu.sync_copy(hb0, acc.at[idx0], add=True)
        plsc.subcore_barrier()
        pltpu.sync_copy(acc.at[pl.ds(r0, _RPT)],
                        out_hbm.at[pl.ds(cid * _NPAD + r0, _RPT)])

    return pl.kernel(
        body,
        mesh=_mesh,
        out_type=[jax.ShapeDtypeStruct((_NC * _NPAD, _D), jnp.float32)],
        scratch_types=[
            pltpu.VMEM((k_,), jnp.int32),
            pltpu.VMEM((k_,), jnp.int32),
            pltpu.VMEM((k_, _D), jnp.float32),
            pltpu.VMEM((k_, _D), jnp.float32),
            pltpu.VMEM_SHARED((_NPAD, _D), jnp.float32),
            pltpu.SemaphoreType.DMA,
            pltpu.SemaphoreType.DMA,
            pltpu.SemaphoreType.DMA,
            pltpu.SemaphoreType.DMA,
        ],
    )


_scatter1 = _make_scatter(_E1, 80, 0)
_scatter2 = _make_scatter(_E2, 32, _E1)


def _p1_body(nf_ref, wa_ref, wb_ref, a_ref, b_ref):
    nf = nf_ref[...]
    a_ref[...] = jnp.dot(nf, wa_ref[...], preferred_element_type=jnp.float32)
    b_ref[...] = jnp.dot(nf, wb_ref[...], preferred_element_type=jnp.float32)


_p1 = pl.pallas_call(
    _p1_body,
    in_specs=[pl.BlockSpec((_N, _D), lambda: (0, 0)),
              pl.BlockSpec((_D, _D), lambda: (0, 0)),
              pl.BlockSpec((_D, _D), lambda: (0, 0))],
    out_specs=[pl.BlockSpec((_N, _D), lambda: (0, 0)),
               pl.BlockSpec((_N, _D), lambda: (0, 0))],
    out_shape=[jax.ShapeDtypeStruct((_N, _D), jnp.float32),
               jax.ShapeDtypeStruct((_N, _D), jnp.float32)],
)


_R = 8000


def _p3_body(g_ref, ef_ref, we_ref, b1_ref, x_ref, st_ref, acc_ref):
    x = (g_ref[...]
         + jnp.dot(ef_ref[...], we_ref[...], preferred_element_type=jnp.float32)
         + b1_ref[...])
    x_ref[...] = x.astype(jnp.bfloat16)
    s = jnp.concatenate([jnp.sum(x, axis=0, keepdims=True),
                         jnp.sum(x * x, axis=0, keepdims=True)], axis=0)

    @pl.when(pl.program_id(0) == 0)
    def _():
        acc_ref[...] = jnp.zeros_like(acc_ref)

    acc_ref[...] += s

    @pl.when(pl.program_id(0) == pl.num_programs(0) - 1)
    def _():
        st_ref[...] = acc_ref[...]


def _make_p3(e_, eoffb_):
    return pl.pallas_call(
        _p3_body,
        grid=(e_ // _R,),
        in_specs=[pl.BlockSpec((_R, _D), lambda i: (i, 0)),
                  pl.BlockSpec((_R, _DE), lambda i: (i + eoffb_, 0)),
                  pl.BlockSpec((_DE, _D), lambda i: (0, 0)),
                  pl.BlockSpec((1, _D), lambda i: (0, 0))],
        out_specs=[pl.BlockSpec((_R, _D), lambda i: (i, 0)),
                   pl.BlockSpec((2, _D), lambda i: (0, 0))],
        out_shape=[jax.ShapeDtypeStruct((e_, _D), jnp.bfloat16),
                   jax.ShapeDtypeStruct((2, _D), jnp.float32)],
        scratch_shapes=[pltpu.VMEM((2, _D), jnp.float32)],
    )


_p3a = _make_p3(_E1, 0)
_p3b = _make_p3(_E2, _E1 // _R)


def _bn_coeffs(st, gamma, beta):
    m = st[0:1] / _E
    v = st[1:2] / _E - m * m
    a = gamma * lax.rsqrt(v + 1e-5)
    return a, beta - m * a


def _p4_body(x_ref, st1_ref, g1c_ref, b1c_ref, st_ref, acc_ref):
    a1, c1 = _bn_coeffs(st1_ref[...], g1c_ref[...], b1c_ref[...])
    f = jax.nn.sigmoid(x_ref[...].astype(jnp.float32) * a1 + c1)
    s = jnp.concatenate([jnp.sum(f, axis=0, keepdims=True),
                         jnp.sum(f * f, axis=0, keepdims=True)], axis=0)

    @pl.when(pl.program_id(0) == 0)
    def _():
        acc_ref[...] = jnp.zeros_like(acc_ref)

    acc_ref[...] += s

    @pl.when(pl.program_id(0) == pl.num_programs(0) - 1)
    def _():
        st_ref[...] = acc_ref[...]


def _make_p4(e_):
    return pl.pallas_call(
        _p4_body,
        grid=(e_ // _R,),
        in_specs=[pl.BlockSpec((_R, _D), lambda i: (i, 0)),
                  pl.BlockSpec((2, _D), lambda i: (0, 0)),
                  pl.BlockSpec((1, _D), lambda i: (0, 0)),
                  pl.BlockSpec((1, _D), lambda i: (0, 0))],
        out_specs=pl.BlockSpec((2, _D), lambda i: (0, 0)),
        out_shape=jax.ShapeDtypeStruct((2, _D), jnp.float32),
        scratch_shapes=[pltpu.VMEM((2, _D), jnp.float32)],
    )


_p4a = _make_p4(_E1)
_p4b = _make_p4(_E2)


def _p5_body(x_ref, st1_ref, g1c_ref, b1c_ref, st2_ref, g2c_ref, b2c_ref,
             h_ref):
    a1, c1 = _bn_coeffs(st1_ref[...], g1c_ref[...], b1c_ref[...])
    a2, c2 = _bn_coeffs(st2_ref[...], g2c_ref[...], b2c_ref[...])
    f = jax.nn.sigmoid(x_ref[...].astype(jnp.float32) * a1 + c1)
    h_ref[...] = f * jax.nn.softplus(f * a2 + c2)


def _make_p5(e_):
    return pl.pallas_call(
        _p5_body,
        grid=(e_ // _R,),
        in_specs=[pl.BlockSpec((_R, _D), lambda i: (i, 0)),
                  pl.BlockSpec((2, _D), lambda i: (0, 0)),
                  pl.BlockSpec((1, _D), lambda i: (0, 0)),
                  pl.BlockSpec((1, _D), lambda i: (0, 0)),
                  pl.BlockSpec((2, _D), lambda i: (0, 0)),
                  pl.BlockSpec((1, _D), lambda i: (0, 0)),
                  pl.BlockSpec((1, _D), lambda i: (0, 0))],
        out_specs=pl.BlockSpec((_R, _D), lambda i: (i, 0)),
        out_shape=jax.ShapeDtypeStruct((e_, _D), jnp.float32),
    )


_p5a = _make_p5(_E1)
_p5b = _make_p5(_E2)


def _p7_body(pa_ref, pb_ref, nf_ref, g3_ref, b3_ref, out_ref):
    nn = (pa_ref[0] + pa_ref[1] + pb_ref[0] + pb_ref[1])[:_N]
    m = jnp.mean(nn, axis=0, keepdims=True)
    v = jnp.mean((nn - m) ** 2, axis=0, keepdims=True)
    out_ref[...] = (nf_ref[...]
                    + g3_ref[...] * (nn - m) * lax.rsqrt(v + 1e-5)
                    + b3_ref[...])


_p7 = pl.pallas_call(
    _p7_body,
    out_shape=jax.ShapeDtypeStruct((_N, _D), jnp.float32),
)


@jax.jit
def kernel(node_feats, edge_feats, edge_index, W1, b1, W2, b2,
           gamma1, beta1, gamma2, beta2, gamma3, beta3):
    src = edge_index[0]
    dst = edge_index[1]
    wa = W1[:_D]
    wb = W1[_D:2 * _D]
    we = W1[2 * _D:]

    a_tab, b_tab = _p1(node_feats, wa, wb)
    (g_a,) = _gather1(a_tab, b_tab, src, dst)
    (g_b,) = _gather2(a_tab, b_tab, src, dst)

    b1c = b1.reshape(1, _D)
    x_a, st1a = _p3a(g_a, edge_feats, we, b1c)
    x_b, st1b = _p3b(g_b, edge_feats, we, b1c)
    st1 = st1a + st1b

    g1c = gamma1.reshape(1, _D)
    b1r = beta1.reshape(1, _D)
    st2 = _p4a(x_a, st1, g1c, b1r) + _p4b(x_b, st1, g1c, b1r)

    g2c = gamma2.reshape(1, _D)
    b2r = beta2.reshape(1, _D)
    h_a = _p5a(x_a, st1, g1c, b1r, st2, g2c, b2r)
    h_b = _p5b(x_b, st1, g1c, b1r, st2, g2c, b2r)

    zeros = jnp.zeros((_NPAD, _D), jnp.float32)
    (pa,) = _scatter1(h_a, dst, zeros)
    (pb,) = _scatter2(h_b, dst, zeros)

    out = _p7(pa.reshape(_NC, _NPAD, _D), pb.reshape(_NC, _NPAD, _D),
              node_feats, gamma3.reshape(1, _D), beta3.reshape(1, _D))
    return (out, edge_feats)
